# baseline (speedup 1.0000x reference)
import functools

import jax
import jax.numpy as jnp
from jax import lax
from jax.experimental import pallas as pl
from jax.experimental.pallas import tpu as pltpu
from jax.experimental.pallas import tpu_sc as plsc


@functools.cache
def _make_gather(V, D, B):
    info = plsc.get_sparse_core_info()
    NC, NS = info.num_cores, info.num_subcores
    L = 16
    NW = NC * NS
    assert D == NW, (D, NW)
    CB = 1024
    NCH = B // CB
    assert B % CB == 0 and CB % L == 0
    mesh = plsc.VectorSubcoreMesh(core_axis_name="c", subcore_axis_name="s")

    @functools.partial(
        pl.kernel,
        mesh=mesh,
        out_type=jax.ShapeDtypeStruct((D, B), jnp.float32),
        scratch_types=[
            pltpu.VMEM((V,), jnp.float32),
            pltpu.VMEM((B,), jnp.int32),
            pltpu.VMEM((2, CB), jnp.float32),
            pltpu.SemaphoreType.DMA,
            pltpu.SemaphoreType.DMA,
            pltpu.SemaphoreType.DMA,
        ],
        compiler_params=pltpu.CompilerParams(
            use_tc_tiling_on_sc=True, needs_layout_passes=False,
            skip_device_barrier=True),
    )
    def k(tableT_hbm, idx_hbm, outT_hbm, col_v, idx_v, out_v, sc, si, so):
        d = lax.axis_index("s") * NC + lax.axis_index("c")
        col_cp = pltpu.async_copy(tableT_hbm.at[d], col_v, sc)
        idx_cp = pltpu.async_copy(idx_hbm, idx_v, si)
        col_cp.wait()
        idx_cp.wait()

        def drain_one():
            pltpu.make_async_copy(
                outT_hbm.at[d, pl.ds(0, CB)], out_v.at[0], so).wait()

        def chunk(c, _):
            bi = lax.rem(c, 2)

            @pl.when(c >= 2)
            def _():
                drain_one()

            @plsc.parallel_loop(0, CB, step=L, unroll=4)
            def body(i):
                out_v[bi, pl.ds(i, L)] = plsc.load_gather(
                    col_v, [idx_v[pl.ds(c * CB + i, L)]])

            pltpu.async_copy(
                out_v.at[bi], outT_hbm.at[d, pl.ds(c * CB, CB)], so)
            return 0

        lax.fori_loop(0, NCH, chunk, 0)
        drain_one()
        drain_one()

    return k


def kernel(domains, table):
    (B,) = domains.shape
    V, D = table.shape
    idx = domains.astype(jnp.int32)
    outT = _make_gather(V, D, B)(table.T, idx)
    return outT.T


# speedup vs baseline: 1.0132x; 1.0132x over previous
import functools

import jax
import jax.numpy as jnp
from jax import lax
from jax.experimental import pallas as pl
from jax.experimental.pallas import tpu as pltpu
from jax.experimental.pallas import tpu_sc as plsc


@functools.cache
def _make_gather(V, D, B):
    info = plsc.get_sparse_core_info()
    NC, NS = info.num_cores, info.num_subcores
    L = 16
    NW = NC * NS
    assert D == NW, (D, NW)
    CB = 2048
    NCH = B // CB
    assert B % CB == 0 and CB % L == 0
    mesh = plsc.VectorSubcoreMesh(core_axis_name="c", subcore_axis_name="s")

    @functools.partial(
        pl.kernel,
        mesh=mesh,
        out_type=jax.ShapeDtypeStruct((D, B), jnp.float32),
        scratch_types=[
            pltpu.VMEM((V,), jnp.float32),
            pltpu.VMEM((B,), jnp.int32),
            pltpu.VMEM((2, CB), jnp.float32),
            pltpu.SemaphoreType.DMA,
            pltpu.SemaphoreType.DMA,
            pltpu.SemaphoreType.DMA,
        ],
        compiler_params=pltpu.CompilerParams(
            use_tc_tiling_on_sc=True, needs_layout_passes=False,
            skip_device_barrier=True),
    )
    def k(tableT_hbm, idx_hbm, outT_hbm, col_v, idx_v, out_v, sc, si, so):
        d = lax.axis_index("s") * NC + lax.axis_index("c")
        col_cp = pltpu.async_copy(tableT_hbm.at[d], col_v, sc)
        idx_cp = pltpu.async_copy(idx_hbm, idx_v, si)
        col_cp.wait()
        idx_cp.wait()

        def drain_one():
            pltpu.make_async_copy(
                outT_hbm.at[d, pl.ds(0, CB)], out_v.at[0], so).wait()

        def chunk(c, _):
            bi = lax.rem(c, 2)

            @pl.when(c >= 2)
            def _():
                drain_one()

            @plsc.parallel_loop(0, CB, step=L, unroll=8)
            def body(i):
                out_v[bi, pl.ds(i, L)] = plsc.load_gather(
                    col_v, [idx_v[pl.ds(c * CB + i, L)]])

            pltpu.async_copy(
                out_v.at[bi], outT_hbm.at[d, pl.ds(c * CB, CB)], so)
            return 0

        lax.fori_loop(0, NCH, chunk, 0)
        drain_one()
        drain_one()

    return k


def kernel(domains, table):
    (B,) = domains.shape
    V, D = table.shape
    idx = domains.astype(jnp.int32)
    outT = _make_gather(V, D, B)(table.T, idx)
    return outT.T
